# Initial kernel scaffold; baseline (speedup 1.0000x reference)
import jax
import jax.numpy as jnp
from jax.experimental import pallas as pl


def kernel(x, W1l, b1, W1r, W2l, b2, W2r):
    raise NotImplementedError("write your pallas kernel here")


# speedup vs baseline: 19.5094x; 19.5094x over previous
import functools

import numpy as np
import jax
import jax.numpy as jnp
from jax.experimental import pallas as pl

C = 96
H = 384
W = 384
N = H * W
OH = 600
OW = 1200
R = 16
NB = H // R
BLK = R * W


def _resize_weights(in_size, out_size):
    inv_scale = in_size / out_size
    sample_f = (np.arange(out_size) + 0.5) * inv_scale - 0.5
    x = np.abs(sample_f[None, :] - np.arange(in_size)[:, None])
    w = np.maximum(0.0, 1.0 - x)
    total = w.sum(axis=0, keepdims=True)
    w = np.where(w > 1e-12, w / total, 0.0)
    return w.astype(np.float32)


def _sage_block(x_ref, up_ref, dn_ref, wl_ref, wr_ref, b_ref, o_ref, *, relu):
    i = pl.program_id(0)
    xc = x_ref[...]
    up = up_ref[0]
    dn = dn_ref[0]
    up = jnp.where(i == 0, 0.0, up)
    dn = jnp.where(i == NB - 1, 0.0, dn)

    above = jnp.concatenate([up, xc[:, : BLK - W]], axis=1)
    below = jnp.concatenate([xc[:, W:], dn], axis=1)
    zcol = jnp.zeros((C, 1), dtype=xc.dtype)
    left = jnp.concatenate([zcol, xc[:, : BLK - 1]], axis=1)
    right = jnp.concatenate([xc[:, 1:], zcol], axis=1)

    lane = jax.lax.broadcasted_iota(jnp.int32, (1, BLK), 1)
    wpos = lane % W
    hpos = i * R + lane // W
    left = jnp.where(wpos == 0, 0.0, left)
    right = jnp.where(wpos == W - 1, 0.0, right)
    agg = (above + below) + (left + right)
    cnt = ((hpos > 0).astype(jnp.float32) + (hpos < H - 1).astype(jnp.float32)
           + (wpos > 0).astype(jnp.float32) + (wpos < W - 1).astype(jnp.float32))
    mean = agg / cnt

    out = (jnp.dot(wl_ref[...], mean, preferred_element_type=jnp.float32)
           + jnp.dot(wr_ref[...], xc, preferred_element_type=jnp.float32)
           + b_ref[...])
    if relu:
        out = jnp.maximum(out, 0.0)
    o_ref[...] = out


def _sage_layer(x2, wl, wr, b, relu):
    x5 = x2.reshape(C, NB, BLK)
    last_rows = jnp.transpose(x5[:, :, BLK - W:], (1, 0, 2))
    first_rows = jnp.transpose(x5[:, :, :W], (1, 0, 2))
    grid = (NB,)
    return pl.pallas_call(
        functools.partial(_sage_block, relu=relu),
        grid=grid,
        in_specs=[
            pl.BlockSpec((C, BLK), lambda i: (0, i)),
            pl.BlockSpec((1, C, W), lambda i: (jnp.maximum(i - 1, 0), 0, 0)),
            pl.BlockSpec((1, C, W), lambda i: (jnp.minimum(i + 1, NB - 1), 0, 0)),
            pl.BlockSpec((C, C), lambda i: (0, 0)),
            pl.BlockSpec((C, C), lambda i: (0, 0)),
            pl.BlockSpec((C, 1), lambda i: (0, 0)),
        ],
        out_specs=pl.BlockSpec((C, BLK), lambda i: (0, i)),
        out_shape=jax.ShapeDtypeStruct((C, N), jnp.float32),
    )(x2, last_rows, first_rows, wl, wr, b)


def _upsample_block(g_ref, a_ref, b_ref, o_ref):
    gc = g_ref[0]
    t = jnp.dot(a_ref[...], gc, preferred_element_type=jnp.float32)
    o_ref[0] = jnp.dot(t, b_ref[...], preferred_element_type=jnp.float32)


def _upsample(g3, a, b):
    return pl.pallas_call(
        _upsample_block,
        grid=(C,),
        in_specs=[
            pl.BlockSpec((1, H, W), lambda c: (c, 0, 0)),
            pl.BlockSpec((OH, H), lambda c: (0, 0)),
            pl.BlockSpec((W, OW), lambda c: (0, 0)),
        ],
        out_specs=pl.BlockSpec((1, OH, OW), lambda c: (c, 0, 0)),
        out_shape=jax.ShapeDtypeStruct((C, OH, OW), jnp.float32),
    )(g3, a, b)


def kernel(x, W1l, b1, W1r, W2l, b2, W2r):
    x2 = x.reshape(C, N)
    h = _sage_layer(x2, W1l, W1r, b1.reshape(C, 1), relu=True)
    g = _sage_layer(h, W2l, W2r, b2.reshape(C, 1), relu=False)
    a = jnp.asarray(_resize_weights(H, OH).T)
    bw = jnp.asarray(_resize_weights(W, OW))
    out = _upsample(g.reshape(C, H, W), a, bw)
    return out.reshape(1, C, OH, OW)


# speedup vs baseline: 25.3232x; 1.2980x over previous
import functools

import numpy as np
import jax
import jax.numpy as jnp
from jax.experimental import pallas as pl

C = 96
H = 384
W = 384
N = H * W
OH = 600
OW = 1200
R = 16
NB = H // R
BLK = R * W


def _resize_weights(in_size, out_size):
    inv_scale = in_size / out_size
    sample_f = (np.arange(out_size) + 0.5) * inv_scale - 0.5
    x = np.abs(sample_f[None, :] - np.arange(in_size)[:, None])
    w = np.maximum(0.0, 1.0 - x)
    total = w.sum(axis=0, keepdims=True)
    w = np.where(w > 1e-12, w / total, 0.0)
    return w.astype(np.float32)


def _sage_block(x_ref, up_ref, dn_ref, wl_ref, wr_ref, b_ref, o_ref, *, relu):
    i = pl.program_id(0)
    xc = x_ref[...]
    up = up_ref[...]
    dn = dn_ref[...]
    up = jnp.where(i == 0, 0.0, up)
    dn = jnp.where(i == NB - 1, 0.0, dn)

    above = jnp.concatenate([up, xc[:, : BLK - W]], axis=1)
    below = jnp.concatenate([xc[:, W:], dn], axis=1)
    zcol = jnp.zeros((C, 1), dtype=xc.dtype)
    left = jnp.concatenate([zcol, xc[:, : BLK - 1]], axis=1)
    right = jnp.concatenate([xc[:, 1:], zcol], axis=1)

    lane = jax.lax.broadcasted_iota(jnp.int32, (1, BLK), 1)
    wpos = lane % W
    hpos = i * R + lane // W
    left = jnp.where(wpos == 0, 0.0, left)
    right = jnp.where(wpos == W - 1, 0.0, right)
    agg = (above + below) + (left + right)
    cnt = ((hpos > 0).astype(jnp.float32) + (hpos < H - 1).astype(jnp.float32)
           + (wpos > 0).astype(jnp.float32) + (wpos < W - 1).astype(jnp.float32))
    mean = agg / cnt

    out = (jnp.dot(wl_ref[...], mean, preferred_element_type=jnp.float32)
           + jnp.dot(wr_ref[...], xc, preferred_element_type=jnp.float32)
           + b_ref[...])
    if relu:
        out = jnp.maximum(out, 0.0)
    o_ref[...] = out


def _sage_layer(x2, wl, wr, b, relu):
    return pl.pallas_call(
        functools.partial(_sage_block, relu=relu),
        grid=(NB,),
        in_specs=[
            pl.BlockSpec((C, BLK), lambda i: (0, i)),
            pl.BlockSpec((C, W), lambda i: (0, jnp.maximum(i * R - 1, 0))),
            pl.BlockSpec((C, W), lambda i: (0, jnp.minimum((i + 1) * R, H - 1))),
            pl.BlockSpec((C, C), lambda i: (0, 0)),
            pl.BlockSpec((C, C), lambda i: (0, 0)),
            pl.BlockSpec((C, 1), lambda i: (0, 0)),
        ],
        out_specs=pl.BlockSpec((C, BLK), lambda i: (0, i)),
        out_shape=jax.ShapeDtypeStruct((C, N), jnp.float32),
    )(x2, x2, x2, wl, wr, b)


def _upsample_block(g_ref, a_ref, b_ref, o_ref):
    gc = g_ref[0]
    t = jnp.dot(a_ref[...], gc, preferred_element_type=jnp.float32)
    o_ref[0] = jnp.dot(t, b_ref[...], preferred_element_type=jnp.float32)


def _upsample(g3, a, b):
    return pl.pallas_call(
        _upsample_block,
        grid=(C,),
        in_specs=[
            pl.BlockSpec((1, H, W), lambda c: (c, 0, 0)),
            pl.BlockSpec((OH, H), lambda c: (0, 0)),
            pl.BlockSpec((W, OW), lambda c: (0, 0)),
        ],
        out_specs=pl.BlockSpec((1, OH, OW), lambda c: (c, 0, 0)),
        out_shape=jax.ShapeDtypeStruct((C, OH, OW), jnp.float32),
    )(g3, a, b)


def kernel(x, W1l, b1, W1r, W2l, b2, W2r):
    x2 = x.reshape(C, N)
    h = _sage_layer(x2, W1l, W1r, b1.reshape(C, 1), relu=True)
    g = _sage_layer(h, W2l, W2r, b2.reshape(C, 1), relu=False)
    a = jnp.asarray(_resize_weights(H, OH).T)
    bw = jnp.asarray(_resize_weights(W, OW))
    out = _upsample(g.reshape(C, H, W), a, bw)
    return out.reshape(1, C, OH, OW)


# speedup vs baseline: 25.6440x; 1.0127x over previous
import numpy as np
import jax
import jax.numpy as jnp
from jax.experimental import pallas as pl

C = 96
H = 384
W = 384
N = H * W
OH = 600
OW = 1200
R = 16
NB = H // R
BLK = R * W
EXT = BLK + 2 * W


def _resize_weights(in_size, out_size):
    inv_scale = in_size / out_size
    sample_f = (np.arange(out_size) + 0.5) * inv_scale - 0.5
    x = np.abs(sample_f[None, :] - np.arange(in_size)[:, None])
    w = np.maximum(0.0, 1.0 - x)
    total = w.sum(axis=0, keepdims=True)
    w = np.where(w > 1e-12, w / total, 0.0)
    return w.astype(np.float32)


def _stencil(xx, base_row):
    L = xx.shape[1] - 2 * W
    above = xx[:, :L]
    below = xx[:, 2 * W:]
    center_l = xx[:, W - 1: W - 1 + L]
    center_r = xx[:, W + 1: W + 1 + L]
    lane = jax.lax.broadcasted_iota(jnp.int32, (1, L), 1)
    wpos = lane % W
    row = (base_row + 1) + lane // W
    left = jnp.where(wpos == 0, 0.0, center_l)
    right = jnp.where(wpos == W - 1, 0.0, center_r)
    agg = (above + below) + (left + right)
    cnt = ((row > 0).astype(jnp.float32) + (row < H - 1).astype(jnp.float32)
           + (wpos > 0).astype(jnp.float32) + (wpos < W - 1).astype(jnp.float32))
    return agg / cnt


def _fused_layers_block(x_ref, up_ref, dn_ref,
                        w1l_ref, w1r_ref, b1_ref, w2l_ref, w2r_ref, b2_ref,
                        o_ref):
    i = pl.program_id(0)
    xc = x_ref[...]
    up2 = jnp.where(i == 0, 0.0, up_ref[...])
    dn2 = jnp.where(i == NB - 1, 0.0, dn_ref[...])
    xx = jnp.concatenate([up2, xc, dn2], axis=1)

    a_row = i * R
    mean1 = _stencil(xx, a_row - 2)
    cen1 = xx[:, W: W + EXT]
    h = (jnp.dot(w1l_ref[...], mean1, preferred_element_type=jnp.float32)
         + jnp.dot(w1r_ref[...], cen1, preferred_element_type=jnp.float32)
         + b1_ref[...])
    h = jnp.maximum(h, 0.0)
    lane = jax.lax.broadcasted_iota(jnp.int32, (1, EXT), 1)
    hrow = (a_row - 1) + lane // W
    h = jnp.where((hrow >= 0) & (hrow < H), h, 0.0)

    mean2 = _stencil(h, a_row - 1)
    cen2 = h[:, W: W + BLK]
    g = (jnp.dot(w2l_ref[...], mean2, preferred_element_type=jnp.float32)
         + jnp.dot(w2r_ref[...], cen2, preferred_element_type=jnp.float32)
         + b2_ref[...])
    o_ref[...] = g


def _fused_layers(x2, w1l, w1r, b1, w2l, w2r, b2):
    return pl.pallas_call(
        _fused_layers_block,
        grid=(NB,),
        in_specs=[
            pl.BlockSpec((C, BLK), lambda i: (0, i)),
            pl.BlockSpec((C, 2 * W), lambda i: (0, jnp.maximum(i * (R // 2) - 1, 0))),
            pl.BlockSpec((C, 2 * W),
                         lambda i: (0, jnp.minimum((i + 1) * (R // 2), N // (2 * W) - 1))),
            pl.BlockSpec((C, C), lambda i: (0, 0)),
            pl.BlockSpec((C, C), lambda i: (0, 0)),
            pl.BlockSpec((C, 1), lambda i: (0, 0)),
            pl.BlockSpec((C, C), lambda i: (0, 0)),
            pl.BlockSpec((C, C), lambda i: (0, 0)),
            pl.BlockSpec((C, 1), lambda i: (0, 0)),
        ],
        out_specs=pl.BlockSpec((C, BLK), lambda i: (0, i)),
        out_shape=jax.ShapeDtypeStruct((C, N), jnp.float32),
    )(x2, x2, x2, w1l, w1r, b1, w2l, w2r, b2)


def _upsample_block(g_ref, a_ref, b_ref, o_ref):
    gc = g_ref[0]
    t = jnp.dot(a_ref[...], gc, preferred_element_type=jnp.float32)
    o_ref[0] = jnp.dot(t, b_ref[...], preferred_element_type=jnp.float32)


def _upsample(g3, a, b):
    return pl.pallas_call(
        _upsample_block,
        grid=(C,),
        in_specs=[
            pl.BlockSpec((1, H, W), lambda c: (c, 0, 0)),
            pl.BlockSpec((OH, H), lambda c: (0, 0)),
            pl.BlockSpec((W, OW), lambda c: (0, 0)),
        ],
        out_specs=pl.BlockSpec((1, OH, OW), lambda c: (c, 0, 0)),
        out_shape=jax.ShapeDtypeStruct((C, OH, OW), jnp.float32),
    )(g3, a, b)


def kernel(x, W1l, b1, W1r, W2l, b2, W2r):
    x2 = x.reshape(C, N)
    g = _fused_layers(x2, W1l, W1r, b1.reshape(C, 1),
                      W2l, W2r, b2.reshape(C, 1))
    a = jnp.asarray(_resize_weights(H, OH).T)
    bw = jnp.asarray(_resize_weights(W, OW))
    out = _upsample(g.reshape(C, H, W), a, bw)
    return out.reshape(1, C, OH, OW)


# speedup vs baseline: 32.7313x; 1.2764x over previous
import numpy as np
import jax
import jax.numpy as jnp
from jax.experimental import pallas as pl

C = 96
H = 384
W = 384
N = H * W
OH = 600
OW = 1200
R = 16
NB = H // R
BLK = R * W
EXT = BLK + 2 * W


def _resize_weights(in_size, out_size):
    inv_scale = in_size / out_size
    sample_f = (np.arange(out_size) + 0.5) * inv_scale - 0.5
    x = np.abs(sample_f[None, :] - np.arange(in_size)[:, None])
    w = np.maximum(0.0, 1.0 - x)
    total = w.sum(axis=0, keepdims=True)
    w = np.where(w > 1e-12, w / total, 0.0)
    return w.astype(np.float32)


def _stencil(xx, base_row):
    L = xx.shape[1] - 2 * W
    above = xx[:, :L]
    below = xx[:, 2 * W:]
    center_l = xx[:, W - 1: W - 1 + L]
    center_r = xx[:, W + 1: W + 1 + L]
    lane = jax.lax.broadcasted_iota(jnp.int32, (1, L), 1)
    wpos = lane % W
    row = (base_row + 1) + lane // W
    left = jnp.where(wpos == 0, 0.0, center_l)
    right = jnp.where(wpos == W - 1, 0.0, center_r)
    agg = (above + below) + (left + right)
    cnt = ((row > 0).astype(jnp.float32) + (row < H - 1).astype(jnp.float32)
           + (wpos > 0).astype(jnp.float32) + (wpos < W - 1).astype(jnp.float32))
    return agg / cnt


def _fused_layers_block(x_ref, up_ref, dn_ref,
                        w1l_ref, w1r_ref, b1_ref, w2l_ref, w2r_ref, b2_ref,
                        o_ref):
    i = pl.program_id(0)
    xc = x_ref[...].reshape(C, BLK)
    up2 = jnp.where(i == 0, 0.0, up_ref[:, 6:8, :].reshape(C, 2 * W))
    dn2 = jnp.where(i == NB - 1, 0.0, dn_ref[:, 0:2, :].reshape(C, 2 * W))
    xx = jnp.concatenate([up2, xc, dn2], axis=1)

    a_row = i * R
    mean1 = _stencil(xx, a_row - 2)
    cen1 = xx[:, W: W + EXT]
    h = (jnp.dot(w1l_ref[...], mean1, preferred_element_type=jnp.float32)
         + jnp.dot(w1r_ref[...], cen1, preferred_element_type=jnp.float32)
         + b1_ref[...])
    h = jnp.maximum(h, 0.0)
    lane = jax.lax.broadcasted_iota(jnp.int32, (1, EXT), 1)
    hrow = (a_row - 1) + lane // W
    h = jnp.where((hrow >= 0) & (hrow < H), h, 0.0)

    mean2 = _stencil(h, a_row - 1)
    cen2 = h[:, W: W + BLK]
    g = (jnp.dot(w2l_ref[...], mean2, preferred_element_type=jnp.float32)
         + jnp.dot(w2r_ref[...], cen2, preferred_element_type=jnp.float32)
         + b2_ref[...])
    o_ref[...] = g.reshape(C, R, W)


def _fused_layers(x3, w1l, w1r, b1, w2l, w2r, b2):
    return pl.pallas_call(
        _fused_layers_block,
        grid=(NB,),
        in_specs=[
            pl.BlockSpec((C, R, W), lambda i: (0, i, 0)),
            pl.BlockSpec((C, 8, W), lambda i: (0, jnp.maximum(i * (R // 8) - 1, 0), 0)),
            pl.BlockSpec((C, 8, W),
                         lambda i: (0, jnp.minimum((i + 1) * (R // 8), H // 8 - 1), 0)),
            pl.BlockSpec((C, C), lambda i: (0, 0)),
            pl.BlockSpec((C, C), lambda i: (0, 0)),
            pl.BlockSpec((C, 1), lambda i: (0, 0)),
            pl.BlockSpec((C, C), lambda i: (0, 0)),
            pl.BlockSpec((C, C), lambda i: (0, 0)),
            pl.BlockSpec((C, 1), lambda i: (0, 0)),
        ],
        out_specs=pl.BlockSpec((C, R, W), lambda i: (0, i, 0)),
        out_shape=jax.ShapeDtypeStruct((C, H, W), jnp.float32),
    )(x3, x3, x3, w1l, w1r, b1, w2l, w2r, b2)


def _upsample_block(g_ref, a_ref, b_ref, o_ref):
    gc = g_ref[0]
    t = jnp.dot(a_ref[...], gc, preferred_element_type=jnp.float32)
    o_ref[0] = jnp.dot(t, b_ref[...], preferred_element_type=jnp.float32)


def _upsample(g3, a, b):
    return pl.pallas_call(
        _upsample_block,
        grid=(C,),
        in_specs=[
            pl.BlockSpec((1, H, W), lambda c: (c, 0, 0)),
            pl.BlockSpec((OH, H), lambda c: (0, 0)),
            pl.BlockSpec((W, OW), lambda c: (0, 0)),
        ],
        out_specs=pl.BlockSpec((1, OH, OW), lambda c: (c, 0, 0)),
        out_shape=jax.ShapeDtypeStruct((C, OH, OW), jnp.float32),
    )(g3, a, b)


def kernel(x, W1l, b1, W1r, W2l, b2, W2r):
    x3 = x.reshape(C, H, W)
    g = _fused_layers(x3, W1l, W1r, b1.reshape(C, 1),
                      W2l, W2r, b2.reshape(C, 1))
    a = jnp.asarray(_resize_weights(H, OH).T)
    bw = jnp.asarray(_resize_weights(W, OW))
    out = _upsample(g, a, bw)
    return out.reshape(1, C, OH, OW)


# speedup vs baseline: 36.7901x; 1.1240x over previous
import numpy as np
import jax
import jax.numpy as jnp
from jax.experimental import pallas as pl

C = 96
H = 384
W = 384
N = H * W
OH = 600
OW = 1200
R = 16
NB = H // R
BLK = R * W
EXT = BLK + 2 * W


def _resize_weights(in_size, out_size):
    inv_scale = in_size / out_size
    sample_f = (np.arange(out_size) + 0.5) * inv_scale - 0.5
    x = np.abs(sample_f[None, :] - np.arange(in_size)[:, None])
    w = np.maximum(0.0, 1.0 - x)
    total = w.sum(axis=0, keepdims=True)
    w = np.where(w > 1e-12, w / total, 0.0)
    return w.astype(np.float32)


def _stencil(xx, base_row):
    L = xx.shape[1] - 2 * W
    above = xx[:, :L]
    below = xx[:, 2 * W:]
    center_l = xx[:, W - 1: W - 1 + L]
    center_r = xx[:, W + 1: W + 1 + L]
    lane = jax.lax.broadcasted_iota(jnp.int32, (1, L), 1)
    wpos = lane % W
    row = (base_row + 1) + lane // W
    left = jnp.where(wpos == 0, 0.0, center_l)
    right = jnp.where(wpos == W - 1, 0.0, center_r)
    agg = (above + below) + (left + right)
    cnt = ((row > 0).astype(jnp.float32) + (row < H - 1).astype(jnp.float32)
           + (wpos > 0).astype(jnp.float32) + (wpos < W - 1).astype(jnp.float32))
    return agg / cnt


def _fused_layers_block(x_ref, up_ref, dn_ref,
                        w1l_ref, w1r_ref, b1_ref, w2l_ref, w2r_ref, b2_ref,
                        o_ref):
    i = pl.program_id(0)
    xc = x_ref[...].reshape(C, BLK)
    up2 = jnp.where(i == 0, 0.0, up_ref[:, 6:8, :].reshape(C, 2 * W))
    dn2 = jnp.where(i == NB - 1, 0.0, dn_ref[:, 0:2, :].reshape(C, 2 * W))
    xx = jnp.concatenate([up2, xc, dn2], axis=1)

    a_row = i * R
    mean1 = _stencil(xx, a_row - 2)
    cen1 = xx[:, W: W + EXT]
    h = (jnp.dot(w1l_ref[...].astype(jnp.bfloat16), mean1.astype(jnp.bfloat16),
                 preferred_element_type=jnp.float32)
         + jnp.dot(w1r_ref[...].astype(jnp.bfloat16), cen1.astype(jnp.bfloat16),
                   preferred_element_type=jnp.float32)
         + b1_ref[...])
    h = jnp.maximum(h, 0.0)
    lane = jax.lax.broadcasted_iota(jnp.int32, (1, EXT), 1)
    hrow = (a_row - 1) + lane // W
    h = jnp.where((hrow >= 0) & (hrow < H), h, 0.0)

    mean2 = _stencil(h, a_row - 1)
    cen2 = h[:, W: W + BLK]
    g = (jnp.dot(w2l_ref[...].astype(jnp.bfloat16), mean2.astype(jnp.bfloat16),
                 preferred_element_type=jnp.float32)
         + jnp.dot(w2r_ref[...].astype(jnp.bfloat16), cen2.astype(jnp.bfloat16),
                   preferred_element_type=jnp.float32)
         + b2_ref[...])
    o_ref[...] = g.reshape(C, R, W)


def _fused_layers(x3, w1l, w1r, b1, w2l, w2r, b2):
    return pl.pallas_call(
        _fused_layers_block,
        grid=(NB,),
        in_specs=[
            pl.BlockSpec((C, R, W), lambda i: (0, i, 0)),
            pl.BlockSpec((C, 8, W), lambda i: (0, jnp.maximum(i * (R // 8) - 1, 0), 0)),
            pl.BlockSpec((C, 8, W),
                         lambda i: (0, jnp.minimum((i + 1) * (R // 8), H // 8 - 1), 0)),
            pl.BlockSpec((C, C), lambda i: (0, 0)),
            pl.BlockSpec((C, C), lambda i: (0, 0)),
            pl.BlockSpec((C, 1), lambda i: (0, 0)),
            pl.BlockSpec((C, C), lambda i: (0, 0)),
            pl.BlockSpec((C, C), lambda i: (0, 0)),
            pl.BlockSpec((C, 1), lambda i: (0, 0)),
        ],
        out_specs=pl.BlockSpec((C, R, W), lambda i: (0, i, 0)),
        out_shape=jax.ShapeDtypeStruct((C, H, W), jnp.float32),
    )(x3, x3, x3, w1l, w1r, b1, w2l, w2r, b2)


CB = 2


def _upsample_block(g_ref, a_ref, b_ref, o_ref):
    for k in range(CB):
        gc = g_ref[k]
        t = jnp.dot(a_ref[...], gc, preferred_element_type=jnp.float32)
        o_ref[k] = jnp.dot(t, b_ref[...], preferred_element_type=jnp.float32)


def _upsample(g3, a, b):
    return pl.pallas_call(
        _upsample_block,
        grid=(C // CB,),
        in_specs=[
            pl.BlockSpec((CB, H, W), lambda c: (c, 0, 0)),
            pl.BlockSpec((OH, H), lambda c: (0, 0)),
            pl.BlockSpec((W, OW), lambda c: (0, 0)),
        ],
        out_specs=pl.BlockSpec((CB, OH, OW), lambda c: (c, 0, 0)),
        out_shape=jax.ShapeDtypeStruct((C, OH, OW), jnp.float32),
    )(g3, a, b)


def kernel(x, W1l, b1, W1r, W2l, b2, W2r):
    x3 = x.reshape(C, H, W)
    g = _fused_layers(x3, W1l, W1r, b1.reshape(C, 1),
                      W2l, W2r, b2.reshape(C, 1))
    a = jnp.asarray(_resize_weights(H, OH).T)
    bw = jnp.asarray(_resize_weights(W, OW))
    out = _upsample(g, a, bw)
    return out.reshape(1, C, OH, OW)


# speedup vs baseline: 38.7178x; 1.0524x over previous
import numpy as np
import jax
import jax.numpy as jnp
from jax.experimental import pallas as pl

C = 96
H = 384
W = 384
N = H * W
OH = 600
OW = 1200
R = 16
NB = H // R
BLK = R * W
EXT = BLK + 2 * W


def _resize_weights(in_size, out_size):
    inv_scale = in_size / out_size
    sample_f = (np.arange(out_size) + 0.5) * inv_scale - 0.5
    x = np.abs(sample_f[None, :] - np.arange(in_size)[:, None])
    w = np.maximum(0.0, 1.0 - x)
    total = w.sum(axis=0, keepdims=True)
    w = np.where(w > 1e-12, w / total, 0.0)
    return w.astype(np.float32)


def _stencil(xx, base_row):
    L = xx.shape[1] - 2 * W
    above = xx[:, :L]
    below = xx[:, 2 * W:]
    center_l = xx[:, W - 1: W - 1 + L]
    center_r = xx[:, W + 1: W + 1 + L]
    lane = jax.lax.broadcasted_iota(jnp.int32, (1, L), 1)
    wpos = lane % W
    row = (base_row + 1) + lane // W
    left = jnp.where(wpos == 0, 0.0, center_l)
    right = jnp.where(wpos == W - 1, 0.0, center_r)
    agg = (above + below) + (left + right)
    cnt = ((row > 0).astype(jnp.float32) + (row < H - 1).astype(jnp.float32)
           + (wpos > 0).astype(jnp.float32) + (wpos < W - 1).astype(jnp.float32))
    return agg / cnt


def _fused_layers_block(x_ref, up_ref, dn_ref,
                        w1l_ref, w1r_ref, b1_ref, w2l_ref, w2r_ref, b2_ref,
                        o_ref):
    i = pl.program_id(0)
    xc = x_ref[...].reshape(C, BLK)
    up2 = jnp.where(i == 0, 0.0, up_ref[:, 6:8, :].reshape(C, 2 * W))
    dn2 = jnp.where(i == NB - 1, 0.0, dn_ref[:, 0:2, :].reshape(C, 2 * W))
    xx = jnp.concatenate([up2, xc, dn2], axis=1)

    a_row = i * R
    mean1 = _stencil(xx, a_row - 2)
    cen1 = xx[:, W: W + EXT]
    h = (jnp.dot(w1l_ref[...].astype(jnp.bfloat16), mean1.astype(jnp.bfloat16),
                 preferred_element_type=jnp.float32)
         + jnp.dot(w1r_ref[...].astype(jnp.bfloat16), cen1.astype(jnp.bfloat16),
                   preferred_element_type=jnp.float32)
         + b1_ref[...])
    h = jnp.maximum(h, 0.0)
    lane = jax.lax.broadcasted_iota(jnp.int32, (1, EXT), 1)
    hrow = (a_row - 1) + lane // W
    h = jnp.where((hrow >= 0) & (hrow < H), h, 0.0)

    mean2 = _stencil(h, a_row - 1)
    cen2 = h[:, W: W + BLK]
    g = (jnp.dot(w2l_ref[...].astype(jnp.bfloat16), mean2.astype(jnp.bfloat16),
                 preferred_element_type=jnp.float32)
         + jnp.dot(w2r_ref[...].astype(jnp.bfloat16), cen2.astype(jnp.bfloat16),
                   preferred_element_type=jnp.float32)
         + b2_ref[...])
    o_ref[...] = g.reshape(C, R, W)


def _fused_layers(x3, w1l, w1r, b1, w2l, w2r, b2):
    return pl.pallas_call(
        _fused_layers_block,
        grid=(NB,),
        in_specs=[
            pl.BlockSpec((C, R, W), lambda i: (0, i, 0)),
            pl.BlockSpec((C, 8, W), lambda i: (0, jnp.maximum(i * (R // 8) - 1, 0), 0)),
            pl.BlockSpec((C, 8, W),
                         lambda i: (0, jnp.minimum((i + 1) * (R // 8), H // 8 - 1), 0)),
            pl.BlockSpec((C, C), lambda i: (0, 0)),
            pl.BlockSpec((C, C), lambda i: (0, 0)),
            pl.BlockSpec((C, 1), lambda i: (0, 0)),
            pl.BlockSpec((C, C), lambda i: (0, 0)),
            pl.BlockSpec((C, C), lambda i: (0, 0)),
            pl.BlockSpec((C, 1), lambda i: (0, 0)),
        ],
        out_specs=pl.BlockSpec((C, R, W), lambda i: (0, i, 0)),
        out_shape=jax.ShapeDtypeStruct((C, H, W), jnp.float32),
    )(x3, x3, x3, w1l, w1r, b1, w2l, w2r, b2)


CB = 4


def _upsample_block(g_ref, a_ref, b_ref, o_ref):
    for k in range(CB):
        gc = g_ref[k]
        t = jnp.dot(a_ref[...], gc, preferred_element_type=jnp.float32)
        o_ref[k] = jnp.dot(t, b_ref[...], preferred_element_type=jnp.float32)


def _upsample(g3, a, b):
    return pl.pallas_call(
        _upsample_block,
        grid=(C // CB,),
        in_specs=[
            pl.BlockSpec((CB, H, W), lambda c: (c, 0, 0)),
            pl.BlockSpec((OH, H), lambda c: (0, 0)),
            pl.BlockSpec((W, OW), lambda c: (0, 0)),
        ],
        out_specs=pl.BlockSpec((CB, OH, OW), lambda c: (c, 0, 0)),
        out_shape=jax.ShapeDtypeStruct((C, OH, OW), jnp.float32),
    )(g3, a, b)


def kernel(x, W1l, b1, W1r, W2l, b2, W2r):
    x3 = x.reshape(C, H, W)
    g = _fused_layers(x3, W1l, W1r, b1.reshape(C, 1),
                      W2l, W2r, b2.reshape(C, 1))
    a = jnp.asarray(_resize_weights(H, OH).T)
    bw = jnp.asarray(_resize_weights(W, OW))
    out = _upsample(g, a, bw)
    return out.reshape(1, C, OH, OW)


# speedup vs baseline: 38.8234x; 1.0027x over previous
import numpy as np
import jax
import jax.numpy as jnp
from jax.experimental import pallas as pl

C = 96
H = 384
W = 384
N = H * W
OH = 600
OW = 1200
R = 32
NB = H // R
BLK = R * W
EXT = BLK + 2 * W


def _resize_weights(in_size, out_size):
    inv_scale = in_size / out_size
    sample_f = (np.arange(out_size) + 0.5) * inv_scale - 0.5
    x = np.abs(sample_f[None, :] - np.arange(in_size)[:, None])
    w = np.maximum(0.0, 1.0 - x)
    total = w.sum(axis=0, keepdims=True)
    w = np.where(w > 1e-12, w / total, 0.0)
    return w.astype(np.float32)


def _stencil(xx, base_row):
    L = xx.shape[1] - 2 * W
    above = xx[:, :L]
    below = xx[:, 2 * W:]
    center_l = xx[:, W - 1: W - 1 + L]
    center_r = xx[:, W + 1: W + 1 + L]
    lane = jax.lax.broadcasted_iota(jnp.int32, (1, L), 1)
    wpos = lane % W
    row = (base_row + 1) + lane // W
    left = jnp.where(wpos == 0, 0.0, center_l)
    right = jnp.where(wpos == W - 1, 0.0, center_r)
    agg = (above + below) + (left + right)
    cnt = ((row > 0).astype(jnp.float32) + (row < H - 1).astype(jnp.float32)
           + (wpos > 0).astype(jnp.float32) + (wpos < W - 1).astype(jnp.float32))
    return agg / cnt


def _fused_layers_block(x_ref, up_ref, dn_ref,
                        w1l_ref, w1r_ref, b1_ref, w2l_ref, w2r_ref, b2_ref,
                        o_ref):
    i = pl.program_id(0)
    xc = x_ref[...].reshape(C, BLK)
    up2 = jnp.where(i == 0, 0.0, up_ref[:, 6:8, :].reshape(C, 2 * W))
    dn2 = jnp.where(i == NB - 1, 0.0, dn_ref[:, 0:2, :].reshape(C, 2 * W))
    xx = jnp.concatenate([up2, xc, dn2], axis=1)

    a_row = i * R
    mean1 = _stencil(xx, a_row - 2)
    cen1 = xx[:, W: W + EXT]
    h = (jnp.dot(w1l_ref[...].astype(jnp.bfloat16), mean1.astype(jnp.bfloat16),
                 preferred_element_type=jnp.float32)
         + jnp.dot(w1r_ref[...].astype(jnp.bfloat16), cen1.astype(jnp.bfloat16),
                   preferred_element_type=jnp.float32)
         + b1_ref[...])
    h = jnp.maximum(h, 0.0)
    lane = jax.lax.broadcasted_iota(jnp.int32, (1, EXT), 1)
    hrow = (a_row - 1) + lane // W
    h = jnp.where((hrow >= 0) & (hrow < H), h, 0.0)

    mean2 = _stencil(h, a_row - 1)
    cen2 = h[:, W: W + BLK]
    g = (jnp.dot(w2l_ref[...].astype(jnp.bfloat16), mean2.astype(jnp.bfloat16),
                 preferred_element_type=jnp.float32)
         + jnp.dot(w2r_ref[...].astype(jnp.bfloat16), cen2.astype(jnp.bfloat16),
                   preferred_element_type=jnp.float32)
         + b2_ref[...])
    o_ref[...] = g.reshape(C, R, W)


def _fused_layers(x3, w1l, w1r, b1, w2l, w2r, b2):
    return pl.pallas_call(
        _fused_layers_block,
        grid=(NB,),
        in_specs=[
            pl.BlockSpec((C, R, W), lambda i: (0, i, 0)),
            pl.BlockSpec((C, 8, W), lambda i: (0, jnp.maximum(i * (R // 8) - 1, 0), 0)),
            pl.BlockSpec((C, 8, W),
                         lambda i: (0, jnp.minimum((i + 1) * (R // 8), H // 8 - 1), 0)),
            pl.BlockSpec((C, C), lambda i: (0, 0)),
            pl.BlockSpec((C, C), lambda i: (0, 0)),
            pl.BlockSpec((C, 1), lambda i: (0, 0)),
            pl.BlockSpec((C, C), lambda i: (0, 0)),
            pl.BlockSpec((C, C), lambda i: (0, 0)),
            pl.BlockSpec((C, 1), lambda i: (0, 0)),
        ],
        out_specs=pl.BlockSpec((C, R, W), lambda i: (0, i, 0)),
        out_shape=jax.ShapeDtypeStruct((C, H, W), jnp.float32),
    )(x3, x3, x3, w1l, w1r, b1, w2l, w2r, b2)


CB = 4


def _upsample_block(g_ref, a_ref, b_ref, o_ref):
    for k in range(CB):
        gc = g_ref[k]
        t = jnp.dot(a_ref[...], gc, preferred_element_type=jnp.float32)
        o_ref[k] = jnp.dot(t, b_ref[...], preferred_element_type=jnp.float32)


def _upsample(g3, a, b):
    return pl.pallas_call(
        _upsample_block,
        grid=(C // CB,),
        in_specs=[
            pl.BlockSpec((CB, H, W), lambda c: (c, 0, 0)),
            pl.BlockSpec((OH, H), lambda c: (0, 0)),
            pl.BlockSpec((W, OW), lambda c: (0, 0)),
        ],
        out_specs=pl.BlockSpec((CB, OH, OW), lambda c: (c, 0, 0)),
        out_shape=jax.ShapeDtypeStruct((C, OH, OW), jnp.float32),
    )(g3, a, b)


def kernel(x, W1l, b1, W1r, W2l, b2, W2r):
    x3 = x.reshape(C, H, W)
    g = _fused_layers(x3, W1l, W1r, b1.reshape(C, 1),
                      W2l, W2r, b2.reshape(C, 1))
    a = jnp.asarray(_resize_weights(H, OH).T)
    bw = jnp.asarray(_resize_weights(W, OW))
    out = _upsample(g, a, bw)
    return out.reshape(1, C, OH, OW)


# speedup vs baseline: 47.2606x; 1.2173x over previous
import numpy as np
import jax
import jax.numpy as jnp
from jax.experimental import pallas as pl

C = 96
H = 384
W = 384
N = H * W
OH = 600
OW = 1200
R = 32
NB = H // R
BLK = R * W
EXT = BLK + 2 * W


def _resize_weights(in_size, out_size):
    inv_scale = in_size / out_size
    sample_f = (np.arange(out_size) + 0.5) * inv_scale - 0.5
    x = np.abs(sample_f[None, :] - np.arange(in_size)[:, None])
    w = np.maximum(0.0, 1.0 - x)
    total = w.sum(axis=0, keepdims=True)
    w = np.where(w > 1e-12, w / total, 0.0)
    return w.astype(np.float32)


def _stencil(xx, base_row):
    L = xx.shape[1] - 2 * W
    above = xx[:, :L]
    below = xx[:, 2 * W:]
    center_l = xx[:, W - 1: W - 1 + L]
    center_r = xx[:, W + 1: W + 1 + L]
    lane = jax.lax.broadcasted_iota(jnp.int32, (1, L), 1)
    wpos = lane % W
    row = (base_row + 1) + lane // W
    left = jnp.where(wpos == 0, 0.0, center_l)
    right = jnp.where(wpos == W - 1, 0.0, center_r)
    agg = (above + below) + (left + right)
    cnt = ((row > 0).astype(jnp.float32) + (row < H - 1).astype(jnp.float32)
           + (wpos > 0).astype(jnp.float32) + (wpos < W - 1).astype(jnp.float32))
    return agg * (1.0 / cnt).astype(agg.dtype)


def _fused_layers_block(x_ref, up_ref, dn_ref,
                        w1l_ref, w1r_ref, b1_ref, w2l_ref, w2r_ref, b2_ref,
                        o_ref):
    i = pl.program_id(0)
    xc = x_ref[...].astype(jnp.bfloat16).reshape(C, BLK)
    up2 = jnp.where(i == 0, 0.0,
                    up_ref[:, 6:8, :].astype(jnp.bfloat16).reshape(C, 2 * W))
    dn2 = jnp.where(i == NB - 1, 0.0,
                    dn_ref[:, 0:2, :].astype(jnp.bfloat16).reshape(C, 2 * W))
    xx = jnp.concatenate([up2, xc, dn2], axis=1)

    a_row = i * R
    mean1 = _stencil(xx, a_row - 2)
    cen1 = xx[:, W: W + EXT]
    h = (jnp.dot(w1l_ref[...].astype(jnp.bfloat16), mean1,
                 preferred_element_type=jnp.float32)
         + jnp.dot(w1r_ref[...].astype(jnp.bfloat16), cen1,
                   preferred_element_type=jnp.float32)
         + b1_ref[...])
    h = jnp.maximum(h, 0.0).astype(jnp.bfloat16)
    lane = jax.lax.broadcasted_iota(jnp.int32, (1, EXT), 1)
    hrow = (a_row - 1) + lane // W
    h = jnp.where((hrow >= 0) & (hrow < H), h, 0.0)

    mean2 = _stencil(h, a_row - 1)
    cen2 = h[:, W: W + BLK]
    g = (jnp.dot(w2l_ref[...].astype(jnp.bfloat16), mean2,
                 preferred_element_type=jnp.float32)
         + jnp.dot(w2r_ref[...].astype(jnp.bfloat16), cen2,
                   preferred_element_type=jnp.float32)
         + b2_ref[...])
    o_ref[...] = g.reshape(C, R, W)


def _fused_layers(x3, w1l, w1r, b1, w2l, w2r, b2):
    return pl.pallas_call(
        _fused_layers_block,
        grid=(NB,),
        in_specs=[
            pl.BlockSpec((C, R, W), lambda i: (0, i, 0)),
            pl.BlockSpec((C, 8, W), lambda i: (0, jnp.maximum(i * (R // 8) - 1, 0), 0)),
            pl.BlockSpec((C, 8, W),
                         lambda i: (0, jnp.minimum((i + 1) * (R // 8), H // 8 - 1), 0)),
            pl.BlockSpec((C, C), lambda i: (0, 0)),
            pl.BlockSpec((C, C), lambda i: (0, 0)),
            pl.BlockSpec((C, 1), lambda i: (0, 0)),
            pl.BlockSpec((C, C), lambda i: (0, 0)),
            pl.BlockSpec((C, C), lambda i: (0, 0)),
            pl.BlockSpec((C, 1), lambda i: (0, 0)),
        ],
        out_specs=pl.BlockSpec((C, R, W), lambda i: (0, i, 0)),
        out_shape=jax.ShapeDtypeStruct((C, H, W), jnp.float32),
    )(x3, x3, x3, w1l, w1r, b1, w2l, w2r, b2)


CB = 4


def _upsample_block(g_ref, a_ref, b_ref, o_ref):
    for k in range(CB):
        gc = g_ref[k]
        t = jnp.dot(a_ref[...], gc, preferred_element_type=jnp.float32)
        o_ref[k] = jnp.dot(t, b_ref[...], preferred_element_type=jnp.float32)


def _upsample(g3, a, b):
    return pl.pallas_call(
        _upsample_block,
        grid=(C // CB,),
        in_specs=[
            pl.BlockSpec((CB, H, W), lambda c: (c, 0, 0)),
            pl.BlockSpec((OH, H), lambda c: (0, 0)),
            pl.BlockSpec((W, OW), lambda c: (0, 0)),
        ],
        out_specs=pl.BlockSpec((CB, OH, OW), lambda c: (c, 0, 0)),
        out_shape=jax.ShapeDtypeStruct((C, OH, OW), jnp.float32),
    )(g3, a, b)


def kernel(x, W1l, b1, W1r, W2l, b2, W2r):
    x3 = x.reshape(C, H, W)
    g = _fused_layers(x3, W1l, W1r, b1.reshape(C, 1),
                      W2l, W2r, b2.reshape(C, 1))
    a = jnp.asarray(_resize_weights(H, OH).T)
    bw = jnp.asarray(_resize_weights(W, OW))
    out = _upsample(g, a, bw)
    return out.reshape(1, C, OH, OW)
